# Initial kernel scaffold; baseline (speedup 1.0000x reference)
import jax
import jax.numpy as jnp
from jax import lax
from jax.experimental import pallas as pl
from jax.experimental.pallas import tpu as pltpu
from jax.experimental.pallas import tpu_sc as plsc

T, N, E, D, H, O = 8, 10000, 160000, 256, 256, 128

NTILES = 16
EPT = E // NTILES
CH = 80
SUPER = 2000
NCH = SUPER // CH
NSUP = EPT // SUPER
NSLOT = 5
RPT = N // NTILES


def _matT(a, b):
    return lax.dot_general(a, b, (((1,), (1,)), ((), ())),
                           preferred_element_type=jnp.float32)


def _sc_body(xtab, ei, agg_out, invc_out, w_out,
             src_big, dst_big, gidxS, dstS, srcS, gath,
             ones16, diag, zrow, z16, cbuf, invflat, wflat, invc_full,
             acc, cnt16, w16, invc_sp,
             gsems, ssems, csem):
    c = lax.axis_index("c")
    s = lax.axis_index("s")
    i16 = jnp.arange(16, dtype=jnp.int32)
    z16i = jnp.zeros((16,), dtype=jnp.int32)
    zf = jnp.zeros((16,), dtype=jnp.float32)
    onef = jnp.ones((16,), dtype=jnp.float32)
    lane0 = i16 < 1

    def _init_row(i, _):
        for g in range(8):
            zrow[i, pl.ds(g * 16, 16)] = zf
        return 0
    lax.fori_loop(0, 125, _init_row, 0)

    def _init16(i, _):
        z16[i] = zf
        return 0
    lax.fori_loop(0, RPT, _init16, 0)

    def _init_ones(i, _):
        ones16[i] = onef
        diag[i] = zf
        return 0
    lax.fori_loop(0, CH, _init_ones, 0)

    rbase = s * RPT

    def _per_t(t, _):
        for k in range(RPT // 125):
            pltpu.sync_copy(zrow, acc.at[pl.ds(rbase + k * 125, 125)])
        pltpu.sync_copy(z16, cnt16.at[pl.ds(rbase, RPT)])
        pltpu.sync_copy(z16, w16.at[pl.ds(rbase, RPT)])
        plsc.subcore_barrier()

        gconst = t * (2 * N) + c

        def _passA(sc, _):
            ebase = s * EPT + sc * SUPER
            pltpu.sync_copy(ei.at[t, 0, pl.ds(ebase, SUPER)], src_big)
            pltpu.sync_copy(ei.at[t, 1, pl.ds(ebase, SUPER)], dst_big)

            def _fill(j, _):
                for g in range(CH // 16):
                    off = j * CH + g * 16
                    sv = src_big[pl.ds(off, 16)]
                    dv = dst_big[pl.ds(off, 16)]
                    gidxS[j, pl.ds(g * 16, 16)] = sv * 2 + gconst
                    dstS[j, pl.ds(g * 16, 16)] = dv
                return 0
            lax.fori_loop(0, NCH, _fill, 0)

            cnt_handles = []
            scat_handles = [None] * NSLOT
            for blk in range(NCH // NSLOT):
                gh = []
                for j2 in range(NSLOT):
                    if blk > 0:
                        scat_handles[j2].wait()
                    ch = blk * NSLOT + j2
                    gh.append(pltpu.async_copy(
                        xtab.at[gidxS.at[ch]], gath.at[j2], gsems[j2]))
                for j2 in range(NSLOT):
                    ch = blk * NSLOT + j2
                    gh[j2].wait()
                    scat_handles[j2] = pltpu.async_copy(
                        gath.at[j2], acc.at[dstS.at[ch]], ssems[j2],
                        add=True)
                    cnt_handles.append(pltpu.async_copy(
                        ones16, cnt16.at[dstS.at[ch]], csem, add=True))
            for hndl in scat_handles:
                hndl.wait()
            for hndl in cnt_handles:
                hndl.wait()
            return 0
        lax.fori_loop(0, NSUP, _passA, 0)
        plsc.subcore_barrier()

        pltpu.sync_copy(cnt16.at[pl.ds(rbase, RPT)], cbuf)

        def _invc(rb, _):
            r0 = rb * 16
            nvalid = jnp.minimum(RPT - r0, 16)
            msk = i16 < nvalid
            idx0 = jnp.where(msk, r0 + i16, 0)
            v = plsc.load_gather(cbuf, [idx0, z16i], mask=msk)
            inv = 1.0 / jnp.maximum(v, 1.0)
            plsc.store_scatter(invflat, [idx0, z16i], inv, mask=msk)
            return 0
        lax.fori_loop(0, (RPT + 15) // 16, _invc, 0)
        pltpu.sync_copy(invflat, invc_sp.at[pl.ds(rbase, RPT)])

        @pl.when(c == 0)
        def _():
            pltpu.sync_copy(invflat, invc_out.at[t, pl.ds(rbase, RPT)])
        plsc.subcore_barrier()

        pltpu.sync_copy(invc_sp, invc_full)

        def _passB(sc, _):
            ebase = s * EPT + sc * SUPER
            pltpu.sync_copy(ei.at[t, 0, pl.ds(ebase, SUPER)], src_big)
            pltpu.sync_copy(ei.at[t, 1, pl.ds(ebase, SUPER)], dst_big)

            def _chunk(j, _):
                for g in range(CH // 16):
                    off = j * CH + g * 16
                    dv = dst_big[pl.ds(off, 16)]
                    srcS[j, pl.ds(g * 16, 16)] = src_big[pl.ds(off, 16)]
                    vals = plsc.load_gather(invc_full, [dv, z16i])
                    plsc.store_scatter(diag, [g * 16 + i16, i16], vals)
                pltpu.sync_copy(diag, w16.at[srcS.at[j]], add=True)
                return 0
            lax.fori_loop(0, NCH, _chunk, 0)
            return 0
        lax.fori_loop(0, NSUP, _passB, 0)
        plsc.subcore_barrier()

        pltpu.sync_copy(w16.at[pl.ds(rbase, RPT)], cbuf)

        def _wsum(r, _):
            v = cbuf[r]
            sval = jnp.sum(v)
            plsc.store_scatter(wflat, [jnp.full((16,), r, jnp.int32), z16i],
                               jnp.full((16,), sval, jnp.float32), mask=lane0)
            return 0
        lax.fori_loop(0, RPT, _wsum, 0)

        @pl.when(c == 0)
        def _():
            pltpu.sync_copy(wflat, w_out.at[t, pl.ds(rbase, RPT)])

        pltpu.sync_copy(acc.at[pl.ds(rbase, RPT)],
                        agg_out.at[t, c, pl.ds(rbase, RPT)])
        plsc.subcore_barrier()
        return 0

    lax.fori_loop(0, T, _per_t, 0)


def _sc_aggregate(xtab, ei):
    mesh = plsc.VectorSubcoreMesh(core_axis_name="c", subcore_axis_name="s")
    f32 = jnp.float32
    kern = pl.kernel(
        _sc_body,
        mesh=mesh,
        out_type=[
            jax.ShapeDtypeStruct((T, 2, N, 128), f32),
            jax.ShapeDtypeStruct((T, N, 1), f32),
            jax.ShapeDtypeStruct((T, N, 1), f32),
        ],
        scratch_types=[
            pltpu.VMEM((SUPER,), jnp.int32),
            pltpu.VMEM((SUPER,), jnp.int32),
            pltpu.VMEM((NCH, CH), jnp.int32),
            pltpu.VMEM((NCH, CH), jnp.int32),
            pltpu.VMEM((NCH, CH), jnp.int32),
            pltpu.VMEM((NSLOT, CH, 128), f32),
            pltpu.VMEM((CH, 16), f32),
            pltpu.VMEM((CH, 16), f32),
            pltpu.VMEM((125, 128), f32),
            pltpu.VMEM((RPT, 16), f32),
            pltpu.VMEM((RPT, 16), f32),
            pltpu.VMEM((RPT, 1), f32),
            pltpu.VMEM((RPT, 1), f32),
            pltpu.VMEM((N, 1), f32),
            pltpu.VMEM_SHARED((N, 128), f32),
            pltpu.VMEM_SHARED((N, 16), f32),
            pltpu.VMEM_SHARED((N, 16), f32),
            pltpu.VMEM_SHARED((N, 1), f32),
            [pltpu.SemaphoreType.DMA] * NSLOT,
            [pltpu.SemaphoreType.DMA] * NSLOT,
            pltpu.SemaphoreType.DMA,
        ],
    )
    return kern(xtab, ei)


_NB = 10
_BN = N // _NB


def _tc_fused_body(x_ref, a0_ref, a1_ref, invc_ref, w_ref,
                   wl_ref, wr_ref, b1_ref, s1_ref, s2_ref):
    nb = pl.program_id(1)
    x = x_ref[0]
    agg = jnp.concatenate([a0_ref[0, 0], a1_ref[0, 0]], axis=1)
    invc = invc_ref[0]
    w = w_ref[0]
    h = _matT(agg * invc, wl_ref[...]) + _matT(x, wr_ref[...]) + b1_ref[...]
    h = jnp.maximum(h, 0.0)
    s1c = jnp.sum(h * w, axis=0, keepdims=True)
    s2c = jnp.sum(h, axis=0, keepdims=True)

    @pl.when(nb == 0)
    def _():
        s1_ref[...] = s1c
        s2_ref[...] = s2c

    @pl.when(nb != 0)
    def _():
        s1_ref[...] += s1c
        s2_ref[...] += s2c


def _tc_fused(x_seq, agg, invc, w, Wl1, Wr1, b1):
    f32 = jnp.float32
    return pl.pallas_call(
        _tc_fused_body,
        grid=(T, _NB),
        in_specs=[
            pl.BlockSpec((1, _BN, D), lambda t, nb: (t, nb, 0)),
            pl.BlockSpec((1, 1, _BN, 128), lambda t, nb: (t, 0, nb, 0)),
            pl.BlockSpec((1, 1, _BN, 128), lambda t, nb: (t, 1, nb, 0)),
            pl.BlockSpec((1, _BN, 1), lambda t, nb: (t, nb, 0)),
            pl.BlockSpec((1, _BN, 1), lambda t, nb: (t, nb, 0)),
            pl.BlockSpec((H, D), lambda t, nb: (0, 0)),
            pl.BlockSpec((H, D), lambda t, nb: (0, 0)),
            pl.BlockSpec((1, H), lambda t, nb: (0, 0)),
        ],
        out_specs=[
            pl.BlockSpec((1, H), lambda t, nb: (t, 0)),
            pl.BlockSpec((1, H), lambda t, nb: (t, 0)),
        ],
        out_shape=[
            jax.ShapeDtypeStruct((T, H), f32),
            jax.ShapeDtypeStruct((T, H), f32),
        ],
    )(x_seq, agg, agg, invc, w, Wl1, Wr1, b1)


def _tc_head_body(s1_ref, s2_ref, wl2_ref, wr2_ref, b2_ref,
                  wih_ref, whh_ref, bih_ref, bhh_ref, wout_ref, bout_ref,
                  out_ref):
    scale = jnp.float32(1.0 / N)
    seq = (_matT(s1_ref[...] * scale, wl2_ref[...])
           + _matT(s2_ref[...] * scale, wr2_ref[...]) + b2_ref[...])
    h = jnp.zeros((1, H), dtype=jnp.float32)
    c = jnp.zeros((1, H), dtype=jnp.float32)
    for t in range(T):
        xt = seq[t:t + 1, :]
        g = (_matT(xt, wih_ref[...]) + bih_ref[...]
             + _matT(h, whh_ref[...]) + bhh_ref[...])
        ig = jax.nn.sigmoid(g[:, 0:H])
        fg = jax.nn.sigmoid(g[:, H:2 * H])
        gg = jnp.tanh(g[:, 2 * H:3 * H])
        og = jax.nn.sigmoid(g[:, 3 * H:4 * H])
        c = fg * c + ig * gg
        h = og * jnp.tanh(c)
    out_ref[...] = _matT(h, wout_ref[...]) + bout_ref[...]


def _tc_head(s1, s2, Wl2, Wr2, b2, W_ih, W_hh, b_ih, b_hh, W_out, b_out):
    return pl.pallas_call(
        _tc_head_body,
        out_shape=jax.ShapeDtypeStruct((1, O), jnp.float32),
    )(s1, s2, Wl2, Wr2, b2, W_ih, W_hh, b_ih, b_hh, W_out, b_out)


def kernel(x_seq, edge_index_seq, Wl1, Wr1, b1, Wl2, Wr2, b2,
           W_ih, W_hh, b_ih, b_hh, W_out, b_out):
    xtab = x_seq.reshape(T * N * 2, 128)
    agg, invc, w = _sc_aggregate(xtab, edge_index_seq)
    s1, s2 = _tc_fused(x_seq, agg, invc, w, Wl1, Wr1, b1[None, :])
    return _tc_head(s1, s2, Wl2, Wr2, b2[None, :],
                    W_ih, W_hh, b_ih[None, :], b_hh[None, :],
                    W_out, b_out[None, :])


# speedup vs baseline: 6.4177x; 6.4177x over previous
import jax
import jax.numpy as jnp
from jax import lax
from jax.experimental import pallas as pl
from jax.experimental.pallas import tpu as pltpu
from jax.experimental.pallas import tpu_sc as plsc

T, N, E, D, H, O = 8, 10000, 160000, 256, 256, 128

NTILES = 16
EPT = E // NTILES
CH = 80
SUPER = 2000
NCH = SUPER // CH
NSUP = EPT // SUPER
NSLOT = 3
RPT = N // NTILES


def _matT(a, b):
    return lax.dot_general(a, b, (((1,), (1,)), ((), ())),
                           preferred_element_type=jnp.float32)


OCH = 624
OTAIL = N - OCH * NTILES


def _sc_body(xtab, ei, agg_out, invc_out, w_out,
             src_big, dst_big, gidxS, dstS, srcS, gath, vbuf,
             ones16, zrow, z16, cbuf,
             acc, cnt16, w16,
             gsems, ssems, csem):
    c = lax.axis_index("c")
    s = lax.axis_index("s")
    zf = jnp.zeros((16,), dtype=jnp.float32)
    onef = jnp.ones((16,), dtype=jnp.float32)

    def _init_row(i, _):
        for g in range(4):
            zrow[i, pl.ds(g * 16, 16)] = zf
        return 0
    lax.fori_loop(0, 25, _init_row, 0)

    def _init16(i, _):
        z16[i] = zf
        return 0
    lax.fori_loop(0, 125, _init16, 0)

    def _init_ones(i, _):
        ones16[i] = onef
        return 0
    lax.fori_loop(0, CH, _init_ones, 0)

    rbase = s * RPT

    def _per_t(t, _):
        def _zero(k, _):
            pltpu.sync_copy(zrow, acc.at[pl.ds(rbase + k * 25, 25)])
            return 0
        lax.fori_loop(0, RPT // 25, _zero, 0)

        def _zero16(k, _):
            pltpu.sync_copy(z16, cnt16.at[pl.ds(rbase + k * 125, 125)])
            pltpu.sync_copy(z16, w16.at[pl.ds(rbase + k * 125, 125)])
            return 0
        lax.fori_loop(0, RPT // 125, _zero16, 0)
        plsc.subcore_barrier()

        def _mk_passA(q, with_cnt):
            def _passA(sc, _):
                ebase = s * EPT + sc * SUPER
                pltpu.sync_copy(ei.at[pl.ds(t * 2 * E + ebase, SUPER)],
                                src_big)
                pltpu.sync_copy(ei.at[pl.ds((t * 2 + 1) * E + ebase, SUPER)],
                                dst_big)
                gconst = t * (4 * N) + q

                def _fill(j, _):
                    for g in range(CH // 16):
                        off = j * CH + g * 16
                        sv = src_big[pl.ds(off, 16)]
                        dv = dst_big[pl.ds(off, 16)]
                        gidxS[j, pl.ds(g * 16, 16)] = sv * 4 + gconst
                        dstS[j, pl.ds(g * 16, 16)] = dv
                    return 0
                lax.fori_loop(0, NCH, _fill, 0)

                cnt_handles = []
                gh = [None] * NSLOT
                sh = [None] * NSLOT
                for step in range(NCH + 1):
                    if step < NCH:
                        slot = step % NSLOT
                        if step >= NSLOT:
                            sh[slot].wait()
                        gh[slot] = pltpu.async_copy(
                            xtab.at[gidxS.at[step]], gath.at[slot],
                            gsems[slot])
                    if step >= 1:
                        ch = step - 1
                        pslot = ch % NSLOT
                        gh[pslot].wait()
                        sh[pslot] = pltpu.async_copy(
                            gath.at[pslot], acc.at[dstS.at[ch]],
                            ssems[pslot], add=True)
                        if with_cnt:
                            cnt_handles.append(pltpu.async_copy(
                                ones16, cnt16.at[dstS.at[ch]], csem,
                                add=True))
                for k in range(NSLOT):
                    sh[(NCH - 1 - k) % NSLOT].wait()
                for hndl in cnt_handles:
                    hndl.wait()
                return 0
            return _passA

        lax.fori_loop(0, NSUP, _mk_passA(c * 2, True), 0)
        plsc.subcore_barrier()

        def _invc_chunk(k, _):
            pltpu.sync_copy(cnt16.at[pl.ds(rbase + k * 125, 125)], cbuf)

            def _invc(r, _):
                cbuf[r] = 1.0 / jnp.maximum(cbuf[r], 1.0)
                return 0
            lax.fori_loop(0, 125, _invc, 0)
            pltpu.sync_copy(cbuf, cnt16.at[pl.ds(rbase + k * 125, 125)])
            return 0
        lax.fori_loop(0, RPT // 125, _invc_chunk, 0)
        plsc.subcore_barrier()

        obase = s * OCH

        @pl.when(c == 0)
        def _():
            pltpu.sync_copy(cnt16.at[pl.ds(obase, OCH)],
                            invc_out.at[t, pl.ds(obase, OCH)])

        @pl.when((c == 0) & (s == NTILES - 1))
        def _():
            pltpu.sync_copy(cnt16.at[pl.ds(OCH * NTILES, OTAIL)],
                            invc_out.at[t, pl.ds(OCH * NTILES, OTAIL)])

        def _aggout(q):
            pltpu.sync_copy(acc.at[pl.ds(obase, OCH)],
                            agg_out.at[t, q, pl.ds(obase, OCH)])

            @pl.when(s == NTILES - 1)
            def _():
                pltpu.sync_copy(acc.at[pl.ds(OCH * NTILES, OTAIL)],
                                agg_out.at[t, q, pl.ds(OCH * NTILES, OTAIL)])

        _aggout(c * 2)
        plsc.subcore_barrier()
        lax.fori_loop(0, RPT // 25, _zero, 0)
        plsc.subcore_barrier()
        lax.fori_loop(0, NSUP, _mk_passA(c * 2 + 1, False), 0)
        plsc.subcore_barrier()
        _aggout(c * 2 + 1)

        def _passB(sc, _):
            ebase = s * EPT + sc * SUPER
            pltpu.sync_copy(ei.at[pl.ds(t * 2 * E + ebase, SUPER)], src_big)
            pltpu.sync_copy(ei.at[pl.ds((t * 2 + 1) * E + ebase, SUPER)],
                            dst_big)

            def _fillb(j, _):
                for g in range(CH // 16):
                    off = j * CH + g * 16
                    srcS[j, pl.ds(g * 16, 16)] = src_big[pl.ds(off, 16)]
                    dstS[j, pl.ds(g * 16, 16)] = dst_big[pl.ds(off, 16)]
                return 0
            lax.fori_loop(0, NCH, _fillb, 0)

            gb = [None] * NSLOT
            wh = [None] * NSLOT
            for step in range(NCH + 1):
                if step < NCH:
                    slot = step % NSLOT
                    if step >= NSLOT:
                        wh[slot].wait()
                    gb[slot] = pltpu.async_copy(
                        cnt16.at[dstS.at[step]], vbuf.at[slot], gsems[slot])
                if step >= 1:
                    ch = step - 1
                    pslot = ch % NSLOT
                    gb[pslot].wait()
                    wh[pslot] = pltpu.async_copy(
                        vbuf.at[pslot], w16.at[srcS.at[ch]], ssems[pslot],
                        add=True)
            for k in range(NSLOT):
                wh[(NCH - 1 - k) % NSLOT].wait()
            return 0
        lax.fori_loop(0, NSUP, _passB, 0)
        plsc.subcore_barrier()

        @pl.when(c == 0)
        def _():
            pltpu.sync_copy(w16.at[pl.ds(obase, OCH)],
                            w_out.at[t, pl.ds(obase, OCH)])

        @pl.when((c == 0) & (s == NTILES - 1))
        def _():
            pltpu.sync_copy(w16.at[pl.ds(OCH * NTILES, OTAIL)],
                            w_out.at[t, pl.ds(OCH * NTILES, OTAIL)])
        plsc.subcore_barrier()
        return 0

    lax.fori_loop(0, T, _per_t, 0)


def _sc_aggregate(xtab, ei):
    mesh = plsc.VectorSubcoreMesh(core_axis_name="c", subcore_axis_name="s",
                                  num_cores=2, num_subcores=NTILES)
    f32 = jnp.float32
    kern = pl.kernel(
        _sc_body,
        mesh=mesh,
        compiler_params=pltpu.CompilerParams(use_tc_tiling_on_sc=False),
        out_type=[
            jax.ShapeDtypeStruct((T, 4, N, 64), f32),
            jax.ShapeDtypeStruct((T, N, 16), f32),
            jax.ShapeDtypeStruct((T, N, 16), f32),
        ],
        scratch_types=[
            pltpu.VMEM((SUPER,), jnp.int32),
            pltpu.VMEM((SUPER,), jnp.int32),
            pltpu.VMEM((NCH, CH), jnp.int32),
            pltpu.VMEM((NCH, CH), jnp.int32),
            pltpu.VMEM((NCH, CH), jnp.int32),
            pltpu.VMEM((NSLOT, CH, 64), f32),
            pltpu.VMEM((NSLOT, CH, 16), f32),
            pltpu.VMEM((CH, 16), f32),
            pltpu.VMEM((25, 64), f32),
            pltpu.VMEM((125, 16), f32),
            pltpu.VMEM((125, 16), f32),
            pltpu.VMEM_SHARED((N, 64), f32),
            pltpu.VMEM_SHARED((N, 16), f32),
            pltpu.VMEM_SHARED((N, 16), f32),
            [pltpu.SemaphoreType.DMA] * NSLOT,
            [pltpu.SemaphoreType.DMA] * NSLOT,
            pltpu.SemaphoreType.DMA,
        ],
    )
    return kern(xtab, ei)


_NB = 5
_BN = N // _NB


def _tc_fused_body(x_ref, a0_ref, a1_ref, a2_ref, a3_ref, invc_ref, w_ref,
                   wl_ref, wr_ref, b1_ref, s1_ref, s2_ref):
    nb = pl.program_id(1)
    x = x_ref[0]
    agg = jnp.concatenate([a0_ref[0, 0], a1_ref[0, 0],
                           a2_ref[0, 0], a3_ref[0, 0]], axis=1)
    invc = invc_ref[0][:, 0:1]
    w = w_ref[0][:, 0:1]
    h = (_matT(agg * invc, wl_ref[...]) + _matT(x, wr_ref[...])
         + b1_ref[...])
    h = jnp.maximum(h, 0.0)
    s1c = jnp.sum(h * w, axis=0, keepdims=True)
    s2c = jnp.sum(h, axis=0, keepdims=True)

    @pl.when(nb == 0)
    def _():
        s1_ref[0] = s1c
        s2_ref[0] = s2c

    @pl.when(nb != 0)
    def _():
        s1_ref[0] += s1c
        s2_ref[0] += s2c


def _tc_fused(x_seq, agg, invc, w, Wl1, Wr1, b1):
    f32 = jnp.float32
    return pl.pallas_call(
        _tc_fused_body,
        grid=(T, _NB),
        in_specs=[
            pl.BlockSpec((1, _BN, D), lambda t, nb: (t, nb, 0)),
            pl.BlockSpec((1, 1, _BN, 64), lambda t, nb: (t, 0, nb, 0)),
            pl.BlockSpec((1, 1, _BN, 64), lambda t, nb: (t, 1, nb, 0)),
            pl.BlockSpec((1, 1, _BN, 64), lambda t, nb: (t, 2, nb, 0)),
            pl.BlockSpec((1, 1, _BN, 64), lambda t, nb: (t, 3, nb, 0)),
            pl.BlockSpec((1, _BN, 16), lambda t, nb: (t, nb, 0)),
            pl.BlockSpec((1, _BN, 16), lambda t, nb: (t, nb, 0)),
            pl.BlockSpec((H, D), lambda t, nb: (0, 0)),
            pl.BlockSpec((H, D), lambda t, nb: (0, 0)),
            pl.BlockSpec((1, H), lambda t, nb: (0, 0)),
        ],
        out_specs=[
            pl.BlockSpec((1, 1, H), lambda t, nb: (t, 0, 0)),
            pl.BlockSpec((1, 1, H), lambda t, nb: (t, 0, 0)),
        ],
        out_shape=[
            jax.ShapeDtypeStruct((T, 1, H), f32),
            jax.ShapeDtypeStruct((T, 1, H), f32),
        ],
    )(x_seq, agg, agg, agg, agg, invc, w, Wl1, Wr1, b1)


def _tc_head_body(s1_ref, s2_ref, wl2_ref, wr2_ref, b2_ref,
                  wih_ref, whh_ref, bih_ref, bhh_ref, wout_ref, bout_ref,
                  out_ref):
    scale = jnp.float32(1.0 / N)
    seq = (_matT(s1_ref[...] * scale, wl2_ref[...])
           + _matT(s2_ref[...] * scale, wr2_ref[...]) + b2_ref[...])
    h = jnp.zeros((1, H), dtype=jnp.float32)
    c = jnp.zeros((1, H), dtype=jnp.float32)
    for t in range(T):
        xt = seq[t:t + 1, :]
        g = (_matT(xt, wih_ref[...]) + bih_ref[...]
             + _matT(h, whh_ref[...]) + bhh_ref[...])
        ig = jax.nn.sigmoid(g[:, 0:H])
        fg = jax.nn.sigmoid(g[:, H:2 * H])
        gg = jnp.tanh(g[:, 2 * H:3 * H])
        og = jax.nn.sigmoid(g[:, 3 * H:4 * H])
        c = fg * c + ig * gg
        h = og * jnp.tanh(c)
    out_ref[...] = _matT(h, wout_ref[...]) + bout_ref[...]


def _tc_head(s1, s2, Wl2, Wr2, b2, W_ih, W_hh, b_ih, b_hh, W_out, b_out):
    return pl.pallas_call(
        _tc_head_body,
        out_shape=jax.ShapeDtypeStruct((1, O), jnp.float32),
    )(s1, s2, Wl2, Wr2, b2, W_ih, W_hh, b_ih, b_hh, W_out, b_out)


def kernel(x_seq, edge_index_seq, Wl1, Wr1, b1, Wl2, Wr2, b2,
           W_ih, W_hh, b_ih, b_hh, W_out, b_out):
    xtab = x_seq.reshape(T * N * 4, 64)
    agg, invc, w = _sc_aggregate(xtab, edge_index_seq.reshape(T * 2 * E))
    s1, s2 = _tc_fused(x_seq, agg, invc, w, Wl1, Wr1, b1[None, :])
    s1 = s1.reshape(T, H)
    s2 = s2.reshape(T, H)
    return _tc_head(s1, s2, Wl2, Wr2, b2[None, :],
                    W_ih, W_hh, b_ih[None, :], b_hh[None, :],
                    W_out, b_out[None, :])


# speedup vs baseline: 9.0201x; 1.4055x over previous
import jax
import jax.numpy as jnp
from jax import lax
from jax.experimental import pallas as pl
from jax.experimental.pallas import tpu as pltpu
from jax.experimental.pallas import tpu_sc as plsc

T, N, E, D, H, O = 8, 10000, 160000, 256, 256, 128

NTILES = 16
EPT = E // NTILES
CH = 80
SUPER = 2000
NCH = SUPER // CH
NSUP = EPT // SUPER
NSLOT = 3
RPT = N // NTILES


def _matT(a, b):
    return lax.dot_general(a, b, (((1,), (1,)), ((), ())),
                           preferred_element_type=jnp.float32)


OCH = 624
OTAIL = N - OCH * NTILES


def _sc_body(xtab, ei, agg_out, invc_out, w_out,
             src_big, dst_big, gidxS, dstS, srcS, gath, vbuf,
             ones16, zrow, z16, cbuf,
             acc, cnt16, w16,
             gsems, ssems, csem):
    c = lax.axis_index("c")
    s = lax.axis_index("s")
    zf = jnp.zeros((16,), dtype=jnp.float32)
    onef = jnp.ones((16,), dtype=jnp.float32)

    zb = jnp.zeros((32,), dtype=jnp.bfloat16)

    def _init_row(i, _):
        for g in range(4):
            zrow[i, pl.ds(g * 32, 32)] = zb
        return 0
    lax.fori_loop(0, 25, _init_row, 0)

    def _init16(i, _):
        z16[i] = zf
        return 0
    lax.fori_loop(0, 125, _init16, 0)

    def _init_ones(i, _):
        ones16[i] = onef
        return 0
    lax.fori_loop(0, CH, _init_ones, 0)

    rbase = s * RPT

    def _per_t(t, _):
        def _zero(k, _):
            pltpu.sync_copy(zrow, acc.at[pl.ds(rbase + k * 25, 25)])
            return 0
        lax.fori_loop(0, RPT // 25, _zero, 0)

        def _zero16(k, _):
            pltpu.sync_copy(z16, cnt16.at[pl.ds(rbase + k * 125, 125)])
            pltpu.sync_copy(z16, w16.at[pl.ds(rbase + k * 125, 125)])
            return 0
        lax.fori_loop(0, RPT // 125, _zero16, 0)
        plsc.subcore_barrier()

        def _passA(sc, _):
            ebase = s * EPT + sc * SUPER
            pltpu.sync_copy(ei.at[pl.ds(t * 2 * E + ebase, SUPER)],
                            src_big)
            pltpu.sync_copy(ei.at[pl.ds((t * 2 + 1) * E + ebase, SUPER)],
                            dst_big)
            gconst = t * (2 * N) + c

            def _fill(j, _):
                for g in range(CH // 16):
                    off = j * CH + g * 16
                    sv = src_big[pl.ds(off, 16)]
                    dv = dst_big[pl.ds(off, 16)]
                    gidxS[j, pl.ds(g * 16, 16)] = sv * 2 + gconst
                    dstS[j, pl.ds(g * 16, 16)] = dv
                return 0
            lax.fori_loop(0, NCH, _fill, 0)

            cnt_handles = []
            gh = [None] * NSLOT
            sh = [None] * NSLOT
            for step in range(NCH + 1):
                if step < NCH:
                    slot = step % NSLOT
                    if step >= NSLOT:
                        sh[slot].wait()
                    gh[slot] = pltpu.async_copy(
                        xtab.at[gidxS.at[step]], gath.at[slot],
                        gsems[slot])
                if step >= 1:
                    ch = step - 1
                    pslot = ch % NSLOT
                    gh[pslot].wait()
                    sh[pslot] = pltpu.async_copy(
                        gath.at[pslot], acc.at[dstS.at[ch]],
                        ssems[pslot], add=True)
                    cnt_handles.append(pltpu.async_copy(
                        ones16, cnt16.at[dstS.at[ch]], csem,
                        add=True))
            for k in range(NSLOT):
                sh[(NCH - 1 - k) % NSLOT].wait()
            for hndl in cnt_handles:
                hndl.wait()
            return 0

        lax.fori_loop(0, NSUP, _passA, 0)
        plsc.subcore_barrier()

        def _invc_chunk(k, _):
            pltpu.sync_copy(cnt16.at[pl.ds(rbase + k * 125, 125)], cbuf)

            def _invc(r, _):
                cbuf[r] = 1.0 / jnp.maximum(cbuf[r], 1.0)
                return 0
            lax.fori_loop(0, 125, _invc, 0)
            pltpu.sync_copy(cbuf, cnt16.at[pl.ds(rbase + k * 125, 125)])
            return 0
        lax.fori_loop(0, RPT // 125, _invc_chunk, 0)
        plsc.subcore_barrier()

        obase = s * OCH

        @pl.when(c == 0)
        def _():
            pltpu.sync_copy(cnt16.at[pl.ds(obase, OCH)],
                            invc_out.at[t, pl.ds(obase, OCH)])

        @pl.when((c == 0) & (s == NTILES - 1))
        def _():
            pltpu.sync_copy(cnt16.at[pl.ds(OCH * NTILES, OTAIL)],
                            invc_out.at[t, pl.ds(OCH * NTILES, OTAIL)])

        pltpu.sync_copy(acc.at[pl.ds(obase, OCH)],
                        agg_out.at[t, c, pl.ds(obase, OCH)])

        @pl.when(s == NTILES - 1)
        def _():
            pltpu.sync_copy(acc.at[pl.ds(OCH * NTILES, OTAIL)],
                            agg_out.at[t, c, pl.ds(OCH * NTILES, OTAIL)])

        def _passB(sc, _):
            ebase = s * EPT + sc * SUPER
            pltpu.sync_copy(ei.at[pl.ds(t * 2 * E + ebase, SUPER)], src_big)
            pltpu.sync_copy(ei.at[pl.ds((t * 2 + 1) * E + ebase, SUPER)],
                            dst_big)

            def _fillb(j, _):
                for g in range(CH // 16):
                    off = j * CH + g * 16
                    srcS[j, pl.ds(g * 16, 16)] = src_big[pl.ds(off, 16)]
                    dstS[j, pl.ds(g * 16, 16)] = dst_big[pl.ds(off, 16)]
                return 0
            lax.fori_loop(0, NCH, _fillb, 0)

            gb = [None] * NSLOT
            wh = [None] * NSLOT
            for step in range(NCH + 1):
                if step < NCH:
                    slot = step % NSLOT
                    if step >= NSLOT:
                        wh[slot].wait()
                    gb[slot] = pltpu.async_copy(
                        cnt16.at[dstS.at[step]], vbuf.at[slot], gsems[slot])
                if step >= 1:
                    ch = step - 1
                    pslot = ch % NSLOT
                    gb[pslot].wait()
                    wh[pslot] = pltpu.async_copy(
                        vbuf.at[pslot], w16.at[srcS.at[ch]], ssems[pslot],
                        add=True)
            for k in range(NSLOT):
                wh[(NCH - 1 - k) % NSLOT].wait()
            return 0
        lax.fori_loop(0, NSUP, _passB, 0)
        plsc.subcore_barrier()

        @pl.when(c == 0)
        def _():
            pltpu.sync_copy(w16.at[pl.ds(obase, OCH)],
                            w_out.at[t, pl.ds(obase, OCH)])

        @pl.when((c == 0) & (s == NTILES - 1))
        def _():
            pltpu.sync_copy(w16.at[pl.ds(OCH * NTILES, OTAIL)],
                            w_out.at[t, pl.ds(OCH * NTILES, OTAIL)])
        plsc.subcore_barrier()
        return 0

    lax.fori_loop(0, T, _per_t, 0)


def _sc_aggregate(xtab, ei):
    mesh = plsc.VectorSubcoreMesh(core_axis_name="c", subcore_axis_name="s",
                                  num_cores=2, num_subcores=NTILES)
    f32 = jnp.float32
    kern = pl.kernel(
        _sc_body,
        mesh=mesh,
        compiler_params=pltpu.CompilerParams(use_tc_tiling_on_sc=False),
        out_type=[
            jax.ShapeDtypeStruct((T, 2, N, 128), jnp.bfloat16),
            jax.ShapeDtypeStruct((T, N, 16), f32),
            jax.ShapeDtypeStruct((T, N, 16), f32),
        ],
        scratch_types=[
            pltpu.VMEM((SUPER,), jnp.int32),
            pltpu.VMEM((SUPER,), jnp.int32),
            pltpu.VMEM((NCH, CH), jnp.int32),
            pltpu.VMEM((NCH, CH), jnp.int32),
            pltpu.VMEM((NCH, CH), jnp.int32),
            pltpu.VMEM((NSLOT, CH, 128), jnp.bfloat16),
            pltpu.VMEM((NSLOT, CH, 16), f32),
            pltpu.VMEM((CH, 16), f32),
            pltpu.VMEM((25, 128), jnp.bfloat16),
            pltpu.VMEM((125, 16), f32),
            pltpu.VMEM((125, 16), f32),
            pltpu.VMEM_SHARED((N, 128), jnp.bfloat16),
            pltpu.VMEM_SHARED((N, 16), f32),
            pltpu.VMEM_SHARED((N, 16), f32),
            [pltpu.SemaphoreType.DMA] * NSLOT,
            [pltpu.SemaphoreType.DMA] * NSLOT,
            pltpu.SemaphoreType.DMA,
        ],
    )
    return kern(xtab, ei)


_NB = 5
_BN = N // _NB


def _tc_fused_body(x_ref, a0_ref, a1_ref, invc_ref, w_ref,
                   wl_ref, wr_ref, b1_ref, s1_ref, s2_ref):
    nb = pl.program_id(1)
    x = x_ref[0]
    agg = jnp.concatenate([a0_ref[0, 0], a1_ref[0, 0]], axis=1)
    invc = invc_ref[0][:, 0:1]
    w = w_ref[0][:, 0:1]
    h = (invc * _matT(agg, wl_ref[...]) + _matT(x, wr_ref[...])
         + b1_ref[...])
    h = jnp.maximum(h, 0.0)
    s1c = jnp.sum(h * w, axis=0, keepdims=True)
    s2c = jnp.sum(h, axis=0, keepdims=True)

    @pl.when(nb == 0)
    def _():
        s1_ref[0] = s1c
        s2_ref[0] = s2c

    @pl.when(nb != 0)
    def _():
        s1_ref[0] += s1c
        s2_ref[0] += s2c


def _tc_fused(x_seq, agg, invc, w, Wl1, Wr1, b1):
    f32 = jnp.float32
    return pl.pallas_call(
        _tc_fused_body,
        grid=(T, _NB),
        in_specs=[
            pl.BlockSpec((1, _BN, D), lambda t, nb: (t, nb, 0)),
            pl.BlockSpec((1, 1, _BN, 128), lambda t, nb: (t, 0, nb, 0)),
            pl.BlockSpec((1, 1, _BN, 128), lambda t, nb: (t, 1, nb, 0)),
            pl.BlockSpec((1, _BN, 16), lambda t, nb: (t, nb, 0)),
            pl.BlockSpec((1, _BN, 16), lambda t, nb: (t, nb, 0)),
            pl.BlockSpec((H, D), lambda t, nb: (0, 0)),
            pl.BlockSpec((H, D), lambda t, nb: (0, 0)),
            pl.BlockSpec((1, H), lambda t, nb: (0, 0)),
        ],
        out_specs=[
            pl.BlockSpec((1, 1, H), lambda t, nb: (t, 0, 0)),
            pl.BlockSpec((1, 1, H), lambda t, nb: (t, 0, 0)),
        ],
        out_shape=[
            jax.ShapeDtypeStruct((T, 1, H), f32),
            jax.ShapeDtypeStruct((T, 1, H), f32),
        ],
    )(x_seq, agg, agg, invc, w, Wl1, Wr1, b1)


def _tc_head_body(s1_ref, s2_ref, wl2_ref, wr2_ref, b2_ref,
                  wih_ref, whh_ref, bih_ref, bhh_ref, wout_ref, bout_ref,
                  out_ref):
    scale = jnp.float32(1.0 / N)
    seq = (_matT(s1_ref[...] * scale, wl2_ref[...])
           + _matT(s2_ref[...] * scale, wr2_ref[...]) + b2_ref[...])
    h = jnp.zeros((1, H), dtype=jnp.float32)
    c = jnp.zeros((1, H), dtype=jnp.float32)
    for t in range(T):
        xt = seq[t:t + 1, :]
        g = (_matT(xt, wih_ref[...]) + bih_ref[...]
             + _matT(h, whh_ref[...]) + bhh_ref[...])
        ig = jax.nn.sigmoid(g[:, 0:H])
        fg = jax.nn.sigmoid(g[:, H:2 * H])
        gg = jnp.tanh(g[:, 2 * H:3 * H])
        og = jax.nn.sigmoid(g[:, 3 * H:4 * H])
        c = fg * c + ig * gg
        h = og * jnp.tanh(c)
    out_ref[...] = _matT(h, wout_ref[...]) + bout_ref[...]


def _tc_head(s1, s2, Wl2, Wr2, b2, W_ih, W_hh, b_ih, b_hh, W_out, b_out):
    return pl.pallas_call(
        _tc_head_body,
        out_shape=jax.ShapeDtypeStruct((1, O), jnp.float32),
    )(s1, s2, Wl2, Wr2, b2, W_ih, W_hh, b_ih, b_hh, W_out, b_out)


def kernel(x_seq, edge_index_seq, Wl1, Wr1, b1, Wl2, Wr2, b2,
           W_ih, W_hh, b_ih, b_hh, W_out, b_out):
    x_bf = x_seq.astype(jnp.bfloat16)
    xtab = x_bf.reshape(T * N * 2, 128)
    agg, invc, w = _sc_aggregate(xtab, edge_index_seq.reshape(T * 2 * E))
    s1, s2 = _tc_fused(x_bf, agg, invc, w,
                       Wl1.astype(jnp.bfloat16), Wr1.astype(jnp.bfloat16),
                       b1[None, :])
    s1 = s1.reshape(T, H)
    s2 = s2.reshape(T, H)
    return _tc_head(s1, s2, Wl2, Wr2, b2[None, :],
                    W_ih, W_hh, b_ih[None, :], b_hh[None, :],
                    W_out, b_out[None, :])


# speedup vs baseline: 9.6653x; 1.0715x over previous
import jax
import jax.numpy as jnp
from jax import lax
from jax.experimental import pallas as pl
from jax.experimental.pallas import tpu as pltpu
from jax.experimental.pallas import tpu_sc as plsc

T, N, E, D, H, O = 8, 10000, 160000, 256, 256, 128

NTILES = 16
EPT = E // NTILES
CH = 80
SUPER = 2000
NCH = SUPER // CH
NSUP = EPT // SUPER
NSLOT = 6
LA = 3
RPT = N // NTILES


def _matT(a, b):
    return lax.dot_general(a, b, (((1,), (1,)), ((), ())),
                           preferred_element_type=jnp.float32)


OCH = 624
OTAIL = N - OCH * NTILES


def _sc_body(xtab, ei, agg_out, invc_out, w_out,
             src_big, dst_big, gidxS, dstS, srcS, gath, vbuf,
             ones16, zrow, z16, cbuf,
             acc, cnt16, w16,
             gsems, ssems, csem):
    c = lax.axis_index("c")
    s = lax.axis_index("s")
    zf = jnp.zeros((16,), dtype=jnp.float32)
    onef = jnp.ones((16,), dtype=jnp.float32)

    zb = jnp.zeros((32,), dtype=jnp.bfloat16)

    def _init_row(i, _):
        for g in range(4):
            zrow[i, pl.ds(g * 32, 32)] = zb
        return 0
    lax.fori_loop(0, 25, _init_row, 0)

    def _init16(i, _):
        z16[i] = zf
        return 0
    lax.fori_loop(0, 125, _init16, 0)

    def _init_ones(i, _):
        ones16[i] = onef
        return 0
    lax.fori_loop(0, CH, _init_ones, 0)

    rbase = s * RPT

    def _per_t(t, _):
        def _zero(k, _):
            pltpu.sync_copy(zrow, acc.at[pl.ds(rbase + k * 25, 25)])
            return 0
        lax.fori_loop(0, RPT // 25, _zero, 0)

        def _zero16(k, _):
            pltpu.sync_copy(z16, cnt16.at[pl.ds(rbase + k * 125, 125)])
            pltpu.sync_copy(z16, w16.at[pl.ds(rbase + k * 125, 125)])
            return 0
        lax.fori_loop(0, RPT // 125, _zero16, 0)
        plsc.subcore_barrier()

        def _passA(sc, _):
            ebase = s * EPT + sc * SUPER
            pltpu.sync_copy(ei.at[pl.ds(t * 2 * E + ebase, SUPER)],
                            src_big)
            pltpu.sync_copy(ei.at[pl.ds((t * 2 + 1) * E + ebase, SUPER)],
                            dst_big)
            gconst = t * (2 * N) + c

            def _fill(j, _):
                for g in range(CH // 16):
                    off = j * CH + g * 16
                    sv = src_big[pl.ds(off, 16)]
                    dv = dst_big[pl.ds(off, 16)]
                    gidxS[j, pl.ds(g * 16, 16)] = sv * 2 + gconst
                    dstS[j, pl.ds(g * 16, 16)] = dv
                return 0
            lax.fori_loop(0, NCH, _fill, 0)

            cnt_handles = []
            gh = [None] * NSLOT
            sh = [None] * NSLOT
            for step in range(NCH + LA):
                if step < NCH:
                    slot = step % NSLOT
                    if step >= NSLOT:
                        sh[slot].wait()
                    gh[slot] = pltpu.async_copy(
                        xtab.at[gidxS.at[step]], gath.at[slot],
                        gsems[slot])
                if step >= LA:
                    ch = step - LA
                    pslot = ch % NSLOT
                    gh[pslot].wait()
                    sh[pslot] = pltpu.async_copy(
                        gath.at[pslot], acc.at[dstS.at[ch]],
                        ssems[pslot], add=True)
                    cnt_handles.append(pltpu.async_copy(
                        ones16, cnt16.at[dstS.at[ch]], csem,
                        add=True))
            for k in range(NSLOT):
                sh[(NCH - 1 - k) % NSLOT].wait()
            for hndl in cnt_handles:
                hndl.wait()
            return 0

        lax.fori_loop(0, NSUP, _passA, 0)
        plsc.subcore_barrier()

        def _invc_chunk(k, _):
            pltpu.sync_copy(cnt16.at[pl.ds(rbase + k * 125, 125)], cbuf)

            def _invc(r, _):
                cbuf[r] = 1.0 / jnp.maximum(cbuf[r], 1.0)
                return 0
            lax.fori_loop(0, 125, _invc, 0)
            pltpu.sync_copy(cbuf, cnt16.at[pl.ds(rbase + k * 125, 125)])
            return 0
        lax.fori_loop(0, RPT // 125, _invc_chunk, 0)
        plsc.subcore_barrier()

        obase = s * OCH

        @pl.when(c == 0)
        def _():
            pltpu.sync_copy(cnt16.at[pl.ds(obase, OCH)],
                            invc_out.at[t, pl.ds(obase, OCH)])

        @pl.when((c == 0) & (s == NTILES - 1))
        def _():
            pltpu.sync_copy(cnt16.at[pl.ds(OCH * NTILES, OTAIL)],
                            invc_out.at[t, pl.ds(OCH * NTILES, OTAIL)])

        pltpu.sync_copy(acc.at[pl.ds(obase, OCH)],
                        agg_out.at[t, c, pl.ds(obase, OCH)])

        @pl.when(s == NTILES - 1)
        def _():
            pltpu.sync_copy(acc.at[pl.ds(OCH * NTILES, OTAIL)],
                            agg_out.at[t, c, pl.ds(OCH * NTILES, OTAIL)])

        def _passB(sc, _):
            ebase = s * EPT + sc * SUPER
            pltpu.sync_copy(ei.at[pl.ds(t * 2 * E + ebase, SUPER)], src_big)
            pltpu.sync_copy(ei.at[pl.ds((t * 2 + 1) * E + ebase, SUPER)],
                            dst_big)

            def _fillb(j, _):
                for g in range(CH // 16):
                    off = j * CH + g * 16
                    srcS[j, pl.ds(g * 16, 16)] = src_big[pl.ds(off, 16)]
                    dstS[j, pl.ds(g * 16, 16)] = dst_big[pl.ds(off, 16)]
                return 0
            lax.fori_loop(0, NCH, _fillb, 0)

            gb = [None] * NSLOT
            wh = [None] * NSLOT
            for step in range(NCH + LA):
                if step < NCH:
                    slot = step % NSLOT
                    if step >= NSLOT:
                        wh[slot].wait()
                    gb[slot] = pltpu.async_copy(
                        cnt16.at[dstS.at[step]], vbuf.at[slot], gsems[slot])
                if step >= LA:
                    ch = step - LA
                    pslot = ch % NSLOT
                    gb[pslot].wait()
                    wh[pslot] = pltpu.async_copy(
                        vbuf.at[pslot], w16.at[srcS.at[ch]], ssems[pslot],
                        add=True)
            for k in range(NSLOT):
                wh[(NCH - 1 - k) % NSLOT].wait()
            return 0
        lax.fori_loop(0, NSUP, _passB, 0)
        plsc.subcore_barrier()

        @pl.when(c == 0)
        def _():
            pltpu.sync_copy(w16.at[pl.ds(obase, OCH)],
                            w_out.at[t, pl.ds(obase, OCH)])

        @pl.when((c == 0) & (s == NTILES - 1))
        def _():
            pltpu.sync_copy(w16.at[pl.ds(OCH * NTILES, OTAIL)],
                            w_out.at[t, pl.ds(OCH * NTILES, OTAIL)])
        plsc.subcore_barrier()
        return 0

    lax.fori_loop(0, T, _per_t, 0)


def _sc_aggregate(xtab, ei):
    mesh = plsc.VectorSubcoreMesh(core_axis_name="c", subcore_axis_name="s",
                                  num_cores=2, num_subcores=NTILES)
    f32 = jnp.float32
    kern = pl.kernel(
        _sc_body,
        mesh=mesh,
        compiler_params=pltpu.CompilerParams(use_tc_tiling_on_sc=False),
        out_type=[
            jax.ShapeDtypeStruct((T, 2, N, 128), jnp.bfloat16),
            jax.ShapeDtypeStruct((T, N, 16), f32),
            jax.ShapeDtypeStruct((T, N, 16), f32),
        ],
        scratch_types=[
            pltpu.VMEM((SUPER,), jnp.int32),
            pltpu.VMEM((SUPER,), jnp.int32),
            pltpu.VMEM((NCH, CH), jnp.int32),
            pltpu.VMEM((NCH, CH), jnp.int32),
            pltpu.VMEM((NCH, CH), jnp.int32),
            pltpu.VMEM((NSLOT, CH, 128), jnp.bfloat16),
            pltpu.VMEM((NSLOT, CH, 16), f32),
            pltpu.VMEM((CH, 16), f32),
            pltpu.VMEM((25, 128), jnp.bfloat16),
            pltpu.VMEM((125, 16), f32),
            pltpu.VMEM((125, 16), f32),
            pltpu.VMEM_SHARED((N, 128), jnp.bfloat16),
            pltpu.VMEM_SHARED((N, 16), f32),
            pltpu.VMEM_SHARED((N, 16), f32),
            [pltpu.SemaphoreType.DMA] * NSLOT,
            [pltpu.SemaphoreType.DMA] * NSLOT,
            pltpu.SemaphoreType.DMA,
        ],
    )
    return kern(xtab, ei)


_NB = 5
_BN = N // _NB


def _tc_fused_body(x_ref, a0_ref, a1_ref, invc_ref, w_ref,
                   wl_ref, wr_ref, b1_ref, s1_ref, s2_ref):
    nb = pl.program_id(1)
    x = x_ref[0]
    agg = jnp.concatenate([a0_ref[0, 0], a1_ref[0, 0]], axis=1)
    invc = invc_ref[0][:, 0:1]
    w = w_ref[0][:, 0:1]
    h = (invc * _matT(agg, wl_ref[...]) + _matT(x, wr_ref[...])
         + b1_ref[...])
    h = jnp.maximum(h, 0.0)
    s1c = jnp.sum(h * w, axis=0, keepdims=True)
    s2c = jnp.sum(h, axis=0, keepdims=True)

    @pl.when(nb == 0)
    def _():
        s1_ref[0] = s1c
        s2_ref[0] = s2c

    @pl.when(nb != 0)
    def _():
        s1_ref[0] += s1c
        s2_ref[0] += s2c


def _tc_fused(x_seq, agg, invc, w, Wl1, Wr1, b1):
    f32 = jnp.float32
    return pl.pallas_call(
        _tc_fused_body,
        grid=(T, _NB),
        in_specs=[
            pl.BlockSpec((1, _BN, D), lambda t, nb: (t, nb, 0)),
            pl.BlockSpec((1, 1, _BN, 128), lambda t, nb: (t, 0, nb, 0)),
            pl.BlockSpec((1, 1, _BN, 128), lambda t, nb: (t, 1, nb, 0)),
            pl.BlockSpec((1, _BN, 16), lambda t, nb: (t, nb, 0)),
            pl.BlockSpec((1, _BN, 16), lambda t, nb: (t, nb, 0)),
            pl.BlockSpec((H, D), lambda t, nb: (0, 0)),
            pl.BlockSpec((H, D), lambda t, nb: (0, 0)),
            pl.BlockSpec((1, H), lambda t, nb: (0, 0)),
        ],
        out_specs=[
            pl.BlockSpec((1, 1, H), lambda t, nb: (t, 0, 0)),
            pl.BlockSpec((1, 1, H), lambda t, nb: (t, 0, 0)),
        ],
        out_shape=[
            jax.ShapeDtypeStruct((T, 1, H), f32),
            jax.ShapeDtypeStruct((T, 1, H), f32),
        ],
    )(x_seq, agg, agg, invc, w, Wl1, Wr1, b1)


def _tc_head_body(s1_ref, s2_ref, wl2_ref, wr2_ref, b2_ref,
                  wih_ref, whh_ref, bih_ref, bhh_ref, wout_ref, bout_ref,
                  out_ref):
    scale = jnp.float32(1.0 / N)
    seq = (_matT(s1_ref[...] * scale, wl2_ref[...])
           + _matT(s2_ref[...] * scale, wr2_ref[...]) + b2_ref[...])
    h = jnp.zeros((1, H), dtype=jnp.float32)
    c = jnp.zeros((1, H), dtype=jnp.float32)
    for t in range(T):
        xt = seq[t:t + 1, :]
        g = (_matT(xt, wih_ref[...]) + bih_ref[...]
             + _matT(h, whh_ref[...]) + bhh_ref[...])
        ig = jax.nn.sigmoid(g[:, 0:H])
        fg = jax.nn.sigmoid(g[:, H:2 * H])
        gg = jnp.tanh(g[:, 2 * H:3 * H])
        og = jax.nn.sigmoid(g[:, 3 * H:4 * H])
        c = fg * c + ig * gg
        h = og * jnp.tanh(c)
    out_ref[...] = _matT(h, wout_ref[...]) + bout_ref[...]


def _tc_head(s1, s2, Wl2, Wr2, b2, W_ih, W_hh, b_ih, b_hh, W_out, b_out):
    return pl.pallas_call(
        _tc_head_body,
        out_shape=jax.ShapeDtypeStruct((1, O), jnp.float32),
    )(s1, s2, Wl2, Wr2, b2, W_ih, W_hh, b_ih, b_hh, W_out, b_out)


def kernel(x_seq, edge_index_seq, Wl1, Wr1, b1, Wl2, Wr2, b2,
           W_ih, W_hh, b_ih, b_hh, W_out, b_out):
    x_bf = x_seq.astype(jnp.bfloat16)
    xtab = x_bf.reshape(T * N * 2, 128)
    agg, invc, w = _sc_aggregate(xtab, edge_index_seq.reshape(T * 2 * E))
    s1, s2 = _tc_fused(x_bf, agg, invc, w,
                       Wl1.astype(jnp.bfloat16), Wr1.astype(jnp.bfloat16),
                       b1[None, :])
    s1 = s1.reshape(T, H)
    s2 = s2.reshape(T, H)
    return _tc_head(s1, s2, Wl2, Wr2, b2[None, :],
                    W_ih, W_hh, b_ih[None, :], b_hh[None, :],
                    W_out, b_out[None, :])


# speedup vs baseline: 10.5501x; 1.0915x over previous
import functools

import jax
import jax.numpy as jnp
from jax import lax
from jax.experimental import pallas as pl
from jax.experimental.pallas import tpu as pltpu
from jax.experimental.pallas import tpu_sc as plsc

T, N, E, D, H, O = 8, 10000, 160000, 256, 256, 128

NTILES = 16
EPT = E // NTILES
CH = 80
SUPER = 2000
NCH = SUPER // CH
NSUP = EPT // SUPER
NSLOT = 6
LA = 3
RPT = N // NTILES


def _matT(a, b):
    return lax.dot_general(a, b, (((1,), (1,)), ((), ())),
                           preferred_element_type=jnp.float32)


OCH = 624
OTAIL = N - OCH * NTILES


TH = T // 2


def _sc_body(t0, xtab, ei, agg_out, invc_out, w_out,
             src_big, dst_big, gidxS, dstS, srcS, gath, vbuf,
             ones16, zrow, z16, cbuf,
             acc, cnt16, w16,
             gsems, ssems, csem):
    c = lax.axis_index("c")
    s = lax.axis_index("s")
    zf = jnp.zeros((16,), dtype=jnp.float32)
    onef = jnp.ones((16,), dtype=jnp.float32)

    zb = jnp.zeros((32,), dtype=jnp.bfloat16)

    def _init_row(i, _):
        for g in range(4):
            zrow[i, pl.ds(g * 32, 32)] = zb
        return 0
    lax.fori_loop(0, 25, _init_row, 0)

    def _init16(i, _):
        z16[i] = zf
        return 0
    lax.fori_loop(0, 125, _init16, 0)

    def _init_ones(i, _):
        ones16[i] = onef
        return 0
    lax.fori_loop(0, CH, _init_ones, 0)

    rbase = s * RPT

    def _per_t(t, _):
        def _zero(k, _):
            pltpu.sync_copy(zrow, acc.at[pl.ds(rbase + k * 25, 25)])
            return 0
        lax.fori_loop(0, RPT // 25, _zero, 0)

        def _zero16(k, _):
            pltpu.sync_copy(z16, cnt16.at[pl.ds(rbase + k * 125, 125)])
            pltpu.sync_copy(z16, w16.at[pl.ds(rbase + k * 125, 125)])
            return 0
        lax.fori_loop(0, RPT // 125, _zero16, 0)
        plsc.subcore_barrier()

        def _passA(sc, _):
            ebase = s * EPT + sc * SUPER
            pltpu.sync_copy(ei.at[pl.ds((t0 + t) * 2 * E + ebase, SUPER)],
                            src_big)
            pltpu.sync_copy(ei.at[pl.ds(((t0 + t) * 2 + 1) * E + ebase, SUPER)],
                            dst_big)
            gconst = (t0 + t) * (2 * N) + c

            def _fill(j, _):
                for g in range(CH // 16):
                    off = j * CH + g * 16
                    sv = src_big[pl.ds(off, 16)]
                    dv = dst_big[pl.ds(off, 16)]
                    gidxS[j, pl.ds(g * 16, 16)] = sv * 2 + gconst
                    dstS[j, pl.ds(g * 16, 16)] = dv
                return 0
            lax.fori_loop(0, NCH, _fill, 0)

            cnt_handles = []
            gh = [None] * NSLOT
            sh = [None] * NSLOT
            for step in range(NCH + LA):
                if step < NCH:
                    slot = step % NSLOT
                    if step >= NSLOT:
                        sh[slot].wait()
                    gh[slot] = pltpu.async_copy(
                        xtab.at[gidxS.at[step]], gath.at[slot],
                        gsems[slot])
                if step >= LA:
                    ch = step - LA
                    pslot = ch % NSLOT
                    gh[pslot].wait()
                    sh[pslot] = pltpu.async_copy(
                        gath.at[pslot], acc.at[dstS.at[ch]],
                        ssems[pslot], add=True)
                    cnt_handles.append(pltpu.async_copy(
                        ones16, cnt16.at[dstS.at[ch]], csem,
                        add=True))
            for k in range(NSLOT):
                sh[(NCH - 1 - k) % NSLOT].wait()
            for hndl in cnt_handles:
                hndl.wait()
            return 0

        lax.fori_loop(0, NSUP, _passA, 0)
        plsc.subcore_barrier()

        def _invc_chunk(k, _):
            pltpu.sync_copy(cnt16.at[pl.ds(rbase + k * 125, 125)], cbuf)

            def _invc(r, _):
                cbuf[r] = 1.0 / jnp.maximum(cbuf[r], 1.0)
                return 0
            lax.fori_loop(0, 125, _invc, 0)
            pltpu.sync_copy(cbuf, cnt16.at[pl.ds(rbase + k * 125, 125)])
            return 0
        lax.fori_loop(0, RPT // 125, _invc_chunk, 0)
        plsc.subcore_barrier()

        obase = s * OCH

        @pl.when(c == 0)
        def _():
            pltpu.sync_copy(cnt16.at[pl.ds(obase, OCH)],
                            invc_out.at[t, pl.ds(obase, OCH)])

        @pl.when((c == 0) & (s == NTILES - 1))
        def _():
            pltpu.sync_copy(cnt16.at[pl.ds(OCH * NTILES, OTAIL)],
                            invc_out.at[t, pl.ds(OCH * NTILES, OTAIL)])

        pltpu.sync_copy(acc.at[pl.ds(obase, OCH)],
                        agg_out.at[t, c, pl.ds(obase, OCH)])

        @pl.when(s == NTILES - 1)
        def _():
            pltpu.sync_copy(acc.at[pl.ds(OCH * NTILES, OTAIL)],
                            agg_out.at[t, c, pl.ds(OCH * NTILES, OTAIL)])

        def _passB(sc, _):
            ebase = s * EPT + sc * SUPER
            pltpu.sync_copy(ei.at[pl.ds((t0 + t) * 2 * E + ebase, SUPER)], src_big)
            pltpu.sync_copy(ei.at[pl.ds(((t0 + t) * 2 + 1) * E + ebase, SUPER)],
                            dst_big)

            def _fillb(j, _):
                for g in range(CH // 16):
                    off = j * CH + g * 16
                    srcS[j, pl.ds(g * 16, 16)] = src_big[pl.ds(off, 16)]
                    dstS[j, pl.ds(g * 16, 16)] = dst_big[pl.ds(off, 16)]
                return 0
            lax.fori_loop(0, NCH, _fillb, 0)

            gb = [None] * NSLOT
            wh = [None] * NSLOT
            for step in range(NCH + LA):
                if step < NCH:
                    slot = step % NSLOT
                    if step >= NSLOT:
                        wh[slot].wait()
                    gb[slot] = pltpu.async_copy(
                        cnt16.at[dstS.at[step]], vbuf.at[slot], gsems[slot])
                if step >= LA:
                    ch = step - LA
                    pslot = ch % NSLOT
                    gb[pslot].wait()
                    wh[pslot] = pltpu.async_copy(
                        vbuf.at[pslot], w16.at[srcS.at[ch]], ssems[pslot],
                        add=True)
            for k in range(NSLOT):
                wh[(NCH - 1 - k) % NSLOT].wait()
            return 0
        lax.fori_loop(0, NSUP, _passB, 0)
        plsc.subcore_barrier()

        @pl.when(c == 0)
        def _():
            pltpu.sync_copy(w16.at[pl.ds(obase, OCH)],
                            w_out.at[t, pl.ds(obase, OCH)])

        @pl.when((c == 0) & (s == NTILES - 1))
        def _():
            pltpu.sync_copy(w16.at[pl.ds(OCH * NTILES, OTAIL)],
                            w_out.at[t, pl.ds(OCH * NTILES, OTAIL)])
        plsc.subcore_barrier()
        return 0

    lax.fori_loop(0, TH, _per_t, 0)


def _sc_aggregate(xtab, ei, t0):
    mesh = plsc.VectorSubcoreMesh(core_axis_name="c", subcore_axis_name="s",
                                  num_cores=2, num_subcores=NTILES)
    f32 = jnp.float32
    kern = pl.kernel(
        functools.partial(_sc_body, t0),
        mesh=mesh,
        compiler_params=pltpu.CompilerParams(use_tc_tiling_on_sc=False),
        out_type=[
            jax.ShapeDtypeStruct((TH, 2, N, 128), jnp.bfloat16),
            jax.ShapeDtypeStruct((TH, N, 16), f32),
            jax.ShapeDtypeStruct((TH, N, 16), f32),
        ],
        scratch_types=[
            pltpu.VMEM((SUPER,), jnp.int32),
            pltpu.VMEM((SUPER,), jnp.int32),
            pltpu.VMEM((NCH, CH), jnp.int32),
            pltpu.VMEM((NCH, CH), jnp.int32),
            pltpu.VMEM((NCH, CH), jnp.int32),
            pltpu.VMEM((NSLOT, CH, 128), jnp.bfloat16),
            pltpu.VMEM((NSLOT, CH, 16), f32),
            pltpu.VMEM((CH, 16), f32),
            pltpu.VMEM((25, 128), jnp.bfloat16),
            pltpu.VMEM((125, 16), f32),
            pltpu.VMEM((125, 16), f32),
            pltpu.VMEM_SHARED((N, 128), jnp.bfloat16),
            pltpu.VMEM_SHARED((N, 16), f32),
            pltpu.VMEM_SHARED((N, 16), f32),
            [pltpu.SemaphoreType.DMA] * NSLOT,
            [pltpu.SemaphoreType.DMA] * NSLOT,
            pltpu.SemaphoreType.DMA,
        ],
    )
    return kern(xtab, ei)


_NB = 5
_BN = N // _NB


def _tc_fused_body(x_ref, a0_ref, a1_ref, invc_ref, w_ref,
                   wl_ref, wr_ref, b1_ref, s1_ref, s2_ref):
    nb = pl.program_id(1)
    x = x_ref[0]
    agg = jnp.concatenate([a0_ref[0, 0], a1_ref[0, 0]], axis=1)
    invc = invc_ref[0][:, 0:1]
    w = w_ref[0][:, 0:1]
    h = (invc * _matT(agg, wl_ref[...]) + _matT(x, wr_ref[...])
         + b1_ref[...])
    h = jnp.maximum(h, 0.0)
    s1c = jnp.sum(h * w, axis=0, keepdims=True)
    s2c = jnp.sum(h, axis=0, keepdims=True)

    @pl.when(nb == 0)
    def _():
        s1_ref[0] = s1c
        s2_ref[0] = s2c

    @pl.when(nb != 0)
    def _():
        s1_ref[0] += s1c
        s2_ref[0] += s2c


def _tc_fused(x_seq, agg, invc, w, Wl1, Wr1, b1):
    f32 = jnp.float32
    return pl.pallas_call(
        _tc_fused_body,
        grid=(TH, _NB),
        in_specs=[
            pl.BlockSpec((1, _BN, D), lambda t, nb: (t, nb, 0)),
            pl.BlockSpec((1, 1, _BN, 128), lambda t, nb: (t, 0, nb, 0)),
            pl.BlockSpec((1, 1, _BN, 128), lambda t, nb: (t, 1, nb, 0)),
            pl.BlockSpec((1, _BN, 16), lambda t, nb: (t, nb, 0)),
            pl.BlockSpec((1, _BN, 16), lambda t, nb: (t, nb, 0)),
            pl.BlockSpec((H, D), lambda t, nb: (0, 0)),
            pl.BlockSpec((H, D), lambda t, nb: (0, 0)),
            pl.BlockSpec((1, H), lambda t, nb: (0, 0)),
        ],
        out_specs=[
            pl.BlockSpec((1, 1, H), lambda t, nb: (t, 0, 0)),
            pl.BlockSpec((1, 1, H), lambda t, nb: (t, 0, 0)),
        ],
        out_shape=[
            jax.ShapeDtypeStruct((TH, 1, H), f32),
            jax.ShapeDtypeStruct((TH, 1, H), f32),
        ],
    )(x_seq, agg, agg, invc, w, Wl1, Wr1, b1)


def _tc_head_body(s1_ref, s2_ref, wl2_ref, wr2_ref, b2_ref,
                  wih_ref, whh_ref, bih_ref, bhh_ref, wout_ref, bout_ref,
                  out_ref):
    scale = jnp.float32(1.0 / N)
    seq = (_matT(s1_ref[...] * scale, wl2_ref[...])
           + _matT(s2_ref[...] * scale, wr2_ref[...]) + b2_ref[...])
    h = jnp.zeros((1, H), dtype=jnp.float32)
    c = jnp.zeros((1, H), dtype=jnp.float32)
    for t in range(T):
        xt = seq[t:t + 1, :]
        g = (_matT(xt, wih_ref[...]) + bih_ref[...]
             + _matT(h, whh_ref[...]) + bhh_ref[...])
        ig = jax.nn.sigmoid(g[:, 0:H])
        fg = jax.nn.sigmoid(g[:, H:2 * H])
        gg = jnp.tanh(g[:, 2 * H:3 * H])
        og = jax.nn.sigmoid(g[:, 3 * H:4 * H])
        c = fg * c + ig * gg
        h = og * jnp.tanh(c)
    out_ref[...] = _matT(h, wout_ref[...]) + bout_ref[...]


def _tc_head(s1, s2, Wl2, Wr2, b2, W_ih, W_hh, b_ih, b_hh, W_out, b_out):
    return pl.pallas_call(
        _tc_head_body,
        out_shape=jax.ShapeDtypeStruct((1, O), jnp.float32),
    )(s1, s2, Wl2, Wr2, b2, W_ih, W_hh, b_ih, b_hh, W_out, b_out)


def kernel(x_seq, edge_index_seq, Wl1, Wr1, b1, Wl2, Wr2, b2,
           W_ih, W_hh, b_ih, b_hh, W_out, b_out):
    x_bf = x_seq.astype(jnp.bfloat16)
    xtab = x_bf.reshape(T * N * 2, 128)
    ei_flat = edge_index_seq.reshape(T * 2 * E)
    wl_bf = Wl1.astype(jnp.bfloat16)
    wr_bf = Wr1.astype(jnp.bfloat16)
    halves = []
    for t0 in (0, TH):
        agg, invc, w = _sc_aggregate(xtab, ei_flat, t0)
        halves.append(_tc_fused(x_bf[t0:t0 + TH], agg, invc, w,
                                wl_bf, wr_bf, b1[None, :]))
    s1 = jnp.concatenate([halves[0][0], halves[1][0]], axis=0).reshape(T, H)
    s2 = jnp.concatenate([halves[0][1], halves[1][1]], axis=0).reshape(T, H)
    return _tc_head(s1, s2, Wl2, Wr2, b2[None, :],
                    W_ih, W_hh, b_ih[None, :], b_hh[None, :],
                    W_out, b_out[None, :])


# speedup vs baseline: 11.0036x; 1.0430x over previous
import functools

import jax
import jax.numpy as jnp
from jax import lax
from jax.experimental import pallas as pl
from jax.experimental.pallas import tpu as pltpu
from jax.experimental.pallas import tpu_sc as plsc

T, N, E, D, H, O = 8, 10000, 160000, 256, 256, 128

NTILES = 16
EPT = E // NTILES
CH = 80
SUPER = 2000
NCH = SUPER // CH
NSUP = EPT // SUPER
NSLOT = 6
LA = 3
RPT = N // NTILES


def _matT(a, b):
    return lax.dot_general(a, b, (((1,), (1,)), ((), ())),
                           preferred_element_type=jnp.float32)


OCH = 624
OTAIL = N - OCH * NTILES


TH = T // 4


def _sc_body(t0, xtab, ei, agg_out, invc_out, w_out,
             src_big, dst_big, gidxS, dstS, srcS, gath, vbuf,
             ones16, zrow, z16, cbuf,
             acc, cnt16, w16,
             gsems, ssems, csem):
    c = lax.axis_index("c")
    s = lax.axis_index("s")
    zf = jnp.zeros((16,), dtype=jnp.float32)
    onef = jnp.ones((16,), dtype=jnp.float32)

    zb = jnp.zeros((32,), dtype=jnp.bfloat16)

    def _init_row(i, _):
        for g in range(4):
            zrow[i, pl.ds(g * 32, 32)] = zb
        return 0
    lax.fori_loop(0, 25, _init_row, 0)

    def _init16(i, _):
        z16[i] = zf
        return 0
    lax.fori_loop(0, 125, _init16, 0)

    def _init_ones(i, _):
        ones16[i] = onef
        return 0
    lax.fori_loop(0, CH, _init_ones, 0)

    rbase = s * RPT

    def _per_t(t, _):
        def _zero(k, _):
            pltpu.sync_copy(zrow, acc.at[pl.ds(rbase + k * 25, 25)])
            return 0
        lax.fori_loop(0, RPT // 25, _zero, 0)

        def _zero16(k, _):
            pltpu.sync_copy(z16, cnt16.at[pl.ds(rbase + k * 125, 125)])
            pltpu.sync_copy(z16, w16.at[pl.ds(rbase + k * 125, 125)])
            return 0
        lax.fori_loop(0, RPT // 125, _zero16, 0)
        plsc.subcore_barrier()

        def _passA(sc, _):
            ebase = s * EPT + sc * SUPER
            pltpu.sync_copy(ei.at[pl.ds((t0 + t) * 2 * E + ebase, SUPER)],
                            src_big)
            pltpu.sync_copy(ei.at[pl.ds(((t0 + t) * 2 + 1) * E + ebase, SUPER)],
                            dst_big)
            gconst = (t0 + t) * (2 * N) + c

            def _fill(j, _):
                for g in range(CH // 16):
                    off = j * CH + g * 16
                    sv = src_big[pl.ds(off, 16)]
                    dv = dst_big[pl.ds(off, 16)]
                    gidxS[j, pl.ds(g * 16, 16)] = sv * 2 + gconst
                    dstS[j, pl.ds(g * 16, 16)] = dv
                return 0
            lax.fori_loop(0, NCH, _fill, 0)

            cnt_handles = []
            gh = [None] * NSLOT
            sh = [None] * NSLOT
            for step in range(NCH + LA):
                if step < NCH:
                    slot = step % NSLOT
                    if step >= NSLOT:
                        sh[slot].wait()
                    gh[slot] = pltpu.async_copy(
                        xtab.at[gidxS.at[step]], gath.at[slot],
                        gsems[slot])
                if step >= LA:
                    ch = step - LA
                    pslot = ch % NSLOT
                    gh[pslot].wait()
                    sh[pslot] = pltpu.async_copy(
                        gath.at[pslot], acc.at[dstS.at[ch]],
                        ssems[pslot], add=True)
                    cnt_handles.append(pltpu.async_copy(
                        ones16, cnt16.at[dstS.at[ch]], csem,
                        add=True))
            for k in range(NSLOT):
                sh[(NCH - 1 - k) % NSLOT].wait()
            for hndl in cnt_handles:
                hndl.wait()
            return 0

        lax.fori_loop(0, NSUP, _passA, 0)
        plsc.subcore_barrier()

        def _invc_chunk(k, _):
            pltpu.sync_copy(cnt16.at[pl.ds(rbase + k * 125, 125)], cbuf)

            def _invc(r, _):
                cbuf[r] = 1.0 / jnp.maximum(cbuf[r], 1.0)
                return 0
            lax.fori_loop(0, 125, _invc, 0)
            pltpu.sync_copy(cbuf, cnt16.at[pl.ds(rbase + k * 125, 125)])
            return 0
        lax.fori_loop(0, RPT // 125, _invc_chunk, 0)
        plsc.subcore_barrier()

        obase = s * OCH

        @pl.when(c == 0)
        def _():
            pltpu.sync_copy(cnt16.at[pl.ds(obase, OCH)],
                            invc_out.at[t, pl.ds(obase, OCH)])

        @pl.when((c == 0) & (s == NTILES - 1))
        def _():
            pltpu.sync_copy(cnt16.at[pl.ds(OCH * NTILES, OTAIL)],
                            invc_out.at[t, pl.ds(OCH * NTILES, OTAIL)])

        pltpu.sync_copy(acc.at[pl.ds(obase, OCH)],
                        agg_out.at[t, c, pl.ds(obase, OCH)])

        @pl.when(s == NTILES - 1)
        def _():
            pltpu.sync_copy(acc.at[pl.ds(OCH * NTILES, OTAIL)],
                            agg_out.at[t, c, pl.ds(OCH * NTILES, OTAIL)])

        def _passB(sc, _):
            ebase = s * EPT + sc * SUPER
            pltpu.sync_copy(ei.at[pl.ds((t0 + t) * 2 * E + ebase, SUPER)], src_big)
            pltpu.sync_copy(ei.at[pl.ds(((t0 + t) * 2 + 1) * E + ebase, SUPER)],
                            dst_big)

            def _fillb(j, _):
                for g in range(CH // 16):
                    off = j * CH + g * 16
                    srcS[j, pl.ds(g * 16, 16)] = src_big[pl.ds(off, 16)]
                    dstS[j, pl.ds(g * 16, 16)] = dst_big[pl.ds(off, 16)]
                return 0
            lax.fori_loop(0, NCH, _fillb, 0)

            gb = [None] * NSLOT
            wh = [None] * NSLOT
            for step in range(NCH + LA):
                if step < NCH:
                    slot = step % NSLOT
                    if step >= NSLOT:
                        wh[slot].wait()
                    gb[slot] = pltpu.async_copy(
                        cnt16.at[dstS.at[step]], vbuf.at[slot], gsems[slot])
                if step >= LA:
                    ch = step - LA
                    pslot = ch % NSLOT
                    gb[pslot].wait()
                    wh[pslot] = pltpu.async_copy(
                        vbuf.at[pslot], w16.at[srcS.at[ch]], ssems[pslot],
                        add=True)
            for k in range(NSLOT):
                wh[(NCH - 1 - k) % NSLOT].wait()
            return 0
        lax.fori_loop(0, NSUP, _passB, 0)
        plsc.subcore_barrier()

        @pl.when(c == 0)
        def _():
            pltpu.sync_copy(w16.at[pl.ds(obase, OCH)],
                            w_out.at[t, pl.ds(obase, OCH)])

        @pl.when((c == 0) & (s == NTILES - 1))
        def _():
            pltpu.sync_copy(w16.at[pl.ds(OCH * NTILES, OTAIL)],
                            w_out.at[t, pl.ds(OCH * NTILES, OTAIL)])
        plsc.subcore_barrier()
        return 0

    lax.fori_loop(0, TH, _per_t, 0)


def _sc_aggregate(xtab, ei, t0):
    mesh = plsc.VectorSubcoreMesh(core_axis_name="c", subcore_axis_name="s",
                                  num_cores=2, num_subcores=NTILES)
    f32 = jnp.float32
    kern = pl.kernel(
        functools.partial(_sc_body, t0),
        mesh=mesh,
        compiler_params=pltpu.CompilerParams(use_tc_tiling_on_sc=False),
        out_type=[
            jax.ShapeDtypeStruct((TH, 2, N, 128), jnp.bfloat16),
            jax.ShapeDtypeStruct((TH, N, 16), f32),
            jax.ShapeDtypeStruct((TH, N, 16), f32),
        ],
        scratch_types=[
            pltpu.VMEM((SUPER,), jnp.int32),
            pltpu.VMEM((SUPER,), jnp.int32),
            pltpu.VMEM((NCH, CH), jnp.int32),
            pltpu.VMEM((NCH, CH), jnp.int32),
            pltpu.VMEM((NCH, CH), jnp.int32),
            pltpu.VMEM((NSLOT, CH, 128), jnp.bfloat16),
            pltpu.VMEM((NSLOT, CH, 16), f32),
            pltpu.VMEM((CH, 16), f32),
            pltpu.VMEM((25, 128), jnp.bfloat16),
            pltpu.VMEM((125, 16), f32),
            pltpu.VMEM((125, 16), f32),
            pltpu.VMEM_SHARED((N, 128), jnp.bfloat16),
            pltpu.VMEM_SHARED((N, 16), f32),
            pltpu.VMEM_SHARED((N, 16), f32),
            [pltpu.SemaphoreType.DMA] * NSLOT,
            [pltpu.SemaphoreType.DMA] * NSLOT,
            pltpu.SemaphoreType.DMA,
        ],
    )
    return kern(xtab, ei)


_NB = 5
_BN = N // _NB


def _tc_fused_body(x_ref, a0_ref, a1_ref, invc_ref, w_ref,
                   wl_ref, wr_ref, b1_ref, s1_ref, s2_ref):
    nb = pl.program_id(1)
    x = x_ref[0]
    agg = jnp.concatenate([a0_ref[0, 0], a1_ref[0, 0]], axis=1)
    invc = invc_ref[0][:, 0:1]
    w = w_ref[0][:, 0:1]
    h = (invc * _matT(agg, wl_ref[...]) + _matT(x, wr_ref[...])
         + b1_ref[...])
    h = jnp.maximum(h, 0.0)
    s1c = jnp.sum(h * w, axis=0, keepdims=True)
    s2c = jnp.sum(h, axis=0, keepdims=True)

    @pl.when(nb == 0)
    def _():
        s1_ref[0] = s1c
        s2_ref[0] = s2c

    @pl.when(nb != 0)
    def _():
        s1_ref[0] += s1c
        s2_ref[0] += s2c


def _tc_fused(x_seq, agg, invc, w, Wl1, Wr1, b1):
    f32 = jnp.float32
    return pl.pallas_call(
        _tc_fused_body,
        grid=(TH, _NB),
        in_specs=[
            pl.BlockSpec((1, _BN, D), lambda t, nb: (t, nb, 0)),
            pl.BlockSpec((1, 1, _BN, 128), lambda t, nb: (t, 0, nb, 0)),
            pl.BlockSpec((1, 1, _BN, 128), lambda t, nb: (t, 1, nb, 0)),
            pl.BlockSpec((1, _BN, 16), lambda t, nb: (t, nb, 0)),
            pl.BlockSpec((1, _BN, 16), lambda t, nb: (t, nb, 0)),
            pl.BlockSpec((H, D), lambda t, nb: (0, 0)),
            pl.BlockSpec((H, D), lambda t, nb: (0, 0)),
            pl.BlockSpec((1, H), lambda t, nb: (0, 0)),
        ],
        out_specs=[
            pl.BlockSpec((1, 1, H), lambda t, nb: (t, 0, 0)),
            pl.BlockSpec((1, 1, H), lambda t, nb: (t, 0, 0)),
        ],
        out_shape=[
            jax.ShapeDtypeStruct((TH, 1, H), f32),
            jax.ShapeDtypeStruct((TH, 1, H), f32),
        ],
    )(x_seq, agg, agg, invc, w, Wl1, Wr1, b1)


def _tc_head_body(s1_ref, s2_ref, wl2_ref, wr2_ref, b2_ref,
                  wih_ref, whh_ref, bih_ref, bhh_ref, wout_ref, bout_ref,
                  out_ref):
    scale = jnp.float32(1.0 / N)
    seq = (_matT(s1_ref[...] * scale, wl2_ref[...])
           + _matT(s2_ref[...] * scale, wr2_ref[...]) + b2_ref[...])
    h = jnp.zeros((1, H), dtype=jnp.float32)
    c = jnp.zeros((1, H), dtype=jnp.float32)
    for t in range(T):
        xt = seq[t:t + 1, :]
        g = (_matT(xt, wih_ref[...]) + bih_ref[...]
             + _matT(h, whh_ref[...]) + bhh_ref[...])
        ig = jax.nn.sigmoid(g[:, 0:H])
        fg = jax.nn.sigmoid(g[:, H:2 * H])
        gg = jnp.tanh(g[:, 2 * H:3 * H])
        og = jax.nn.sigmoid(g[:, 3 * H:4 * H])
        c = fg * c + ig * gg
        h = og * jnp.tanh(c)
    out_ref[...] = _matT(h, wout_ref[...]) + bout_ref[...]


def _tc_head(s1, s2, Wl2, Wr2, b2, W_ih, W_hh, b_ih, b_hh, W_out, b_out):
    return pl.pallas_call(
        _tc_head_body,
        out_shape=jax.ShapeDtypeStruct((1, O), jnp.float32),
    )(s1, s2, Wl2, Wr2, b2, W_ih, W_hh, b_ih, b_hh, W_out, b_out)


def kernel(x_seq, edge_index_seq, Wl1, Wr1, b1, Wl2, Wr2, b2,
           W_ih, W_hh, b_ih, b_hh, W_out, b_out):
    x_bf = x_seq.astype(jnp.bfloat16)
    xtab = x_bf.reshape(T * N * 2, 128)
    ei_flat = edge_index_seq.reshape(T * 2 * E)
    wl_bf = Wl1.astype(jnp.bfloat16)
    wr_bf = Wr1.astype(jnp.bfloat16)
    parts = []
    for t0 in range(0, T, TH):
        agg, invc, w = _sc_aggregate(xtab, ei_flat, t0)
        parts.append(_tc_fused(x_bf[t0:t0 + TH], agg, invc, w,
                               wl_bf, wr_bf, b1[None, :]))
    s1 = jnp.concatenate([pp[0] for pp in parts], axis=0).reshape(T, H)
    s2 = jnp.concatenate([pp[1] for pp in parts], axis=0).reshape(T, H)
    return _tc_head(s1, s2, Wl2, Wr2, b2[None, :],
                    W_ih, W_hh, b_ih[None, :], b_hh[None, :],
                    W_out, b_out[None, :])
